# baseline (speedup 1.0000x reference)
import functools

import jax
import jax.numpy as jnp
from jax.experimental import pallas as pl
from jax.experimental.pallas import tpu as pltpu

_VMEM_LIMIT = 58 * 1024 * 1024
_ROW_TILE = 512


def _gcn_kernel(adj_ref, x_ref, w0_ref, w1_ref, w2_ref, b0_ref, b1_ref,
                b2_ref, o_ref, abf, ua, ub, dsc, acc, *, f_out):
    s = pl.program_id(0)
    m = pl.program_id(1)
    tm = o_ref.shape[0]
    r0 = pl.multiple_of(m * tm, tm)
    rows = pl.ds(r0, tm)

    @pl.when(s == 0)
    def _prep():
        a = adj_ref[...]
        col = jax.lax.broadcasted_iota(jnp.int32, a.shape, 1)
        row = jax.lax.broadcasted_iota(jnp.int32, a.shape, 0) + r0
        a_bf = jnp.where(col == row, a + 1.0, a).astype(jnp.bfloat16)
        abf[rows, :] = a_bf
        d = jax.lax.rsqrt(jnp.sum(a, axis=1, keepdims=True) + 1.0)
        dsc[rows, :] = d
        z = jnp.dot(x_ref[...].astype(jnp.bfloat16),
                    w0_ref[...].astype(jnp.bfloat16),
                    preferred_element_type=jnp.float32)
        u0 = (d * z).astype(jnp.bfloat16)
        part = jax.lax.dot_general(
            a_bf, u0, (((0,), (0,)), ((), ())),
            preferred_element_type=jnp.float32)

        @pl.when(m == 0)
        def _():
            acc[...] = part

        @pl.when(m > 0)
        def _():
            acc[...] += part

    @pl.when(s == 1)
    def _layer0():
        d = dsc[rows, :]
        out = d * acc[rows, :] + b0_ref[...]
        h = jnp.maximum(out, 0.0).astype(jnp.bfloat16)
        z = jnp.dot(h, w1_ref[...].astype(jnp.bfloat16),
                    preferred_element_type=jnp.float32)
        ua[rows, :] = (d * z).astype(jnp.bfloat16)

    def _transform1(mm):
        rp = pl.ds(pl.multiple_of(mm * tm, tm), tm)
        d = dsc[rp, :]
        out = d * acc[rp, :] + b1_ref[...]
        h = jnp.maximum(out, 0.0).astype(jnp.bfloat16)
        z = jnp.dot(h, w2_ref[...].astype(jnp.bfloat16),
                    preferred_element_type=jnp.float32)
        ub[rp, :f_out] = (d * z).astype(jnp.bfloat16)

    @pl.when(s == 2)
    def _layer1():
        acc[rows, :] = jnp.dot(abf[rows, :], ua[...],
                               preferred_element_type=jnp.float32)

        @pl.when(m > 0)
        def _():
            _transform1(m - 1)

    @pl.when(s == 3)
    def _layer2():
        @pl.when(m == 0)
        def _():
            _transform1(pl.num_programs(1) - 1)

        agg = jnp.dot(abf[rows, :], ub[:, :f_out],
                      preferred_element_type=jnp.float32)
        o_ref[...] = dsc[rows, :] * agg + b2_ref[...]


def kernel(x, adj, w_0, b_0, w_1, b_1, w_2, b_2):
    n, f_in = x.shape
    f_h = w_1.shape[0]
    f_out = w_2.shape[1]
    tm = _ROW_TILE
    nstrips = n // tm
    num_stages = 4

    last = nstrips - 1
    adj_idx = lambda s, m: (jax.lax.select(s == 0, m, last), 0)
    out_idx = lambda s, m: (jax.lax.select(s == num_stages - 1, m, 0), 0)
    const = lambda s, m: (0, 0)

    return pl.pallas_call(
        functools.partial(_gcn_kernel, f_out=f_out),
        grid=(num_stages, nstrips),
        in_specs=[
            pl.BlockSpec((tm, n), adj_idx),
            pl.BlockSpec((tm, f_in), adj_idx),
            pl.BlockSpec((f_in, f_h), const),
            pl.BlockSpec((f_h, f_h), const),
            pl.BlockSpec((f_h, f_out), const),
            pl.BlockSpec((1, f_h), const),
            pl.BlockSpec((1, f_h), const),
            pl.BlockSpec((1, f_out), const),
        ],
        out_specs=pl.BlockSpec((tm, f_out), out_idx),
        out_shape=jax.ShapeDtypeStruct((n, f_out), jnp.float32),
        scratch_shapes=[
            pltpu.VMEM((n, n), jnp.bfloat16),
            pltpu.VMEM((n, f_h), jnp.bfloat16),
            pltpu.VMEM((n, f_h), jnp.bfloat16),
            pltpu.VMEM((n, 1), jnp.float32),
            pltpu.VMEM((n, f_h), jnp.float32),
        ],
        compiler_params=pltpu.CompilerParams(
            dimension_semantics=("arbitrary", "arbitrary"),
            vmem_limit_bytes=_VMEM_LIMIT,
        ),
    )(adj, x, w_0, w_1, w_2,
      b_0.reshape(1, -1), b_1.reshape(1, -1), b_2.reshape(1, -1))


# speedup vs baseline: 1.0187x; 1.0187x over previous
import functools

import jax
import jax.numpy as jnp
from jax.experimental import pallas as pl
from jax.experimental.pallas import tpu as pltpu

_VMEM_LIMIT = 58 * 1024 * 1024
_ROW_TILE = 512


def _gcn_kernel(adj_ref, x_ref, w0_ref, w1_ref, w2_ref, b0_ref, b1_ref,
                b2_ref, o_ref, abf, ua, ub, dsc, acc, *, f_out):
    s = pl.program_id(0)
    m = pl.program_id(1)
    tm = o_ref.shape[0]
    r0 = pl.multiple_of(m * tm, tm)
    rows = pl.ds(r0, tm)

    @pl.when(s == 0)
    def _prep():
        a = adj_ref[...]
        col = jax.lax.broadcasted_iota(jnp.int32, a.shape, 1)
        row = jax.lax.broadcasted_iota(jnp.int32, a.shape, 0) + r0
        a_bf = jnp.where(col == row, a + 1.0, a).astype(jnp.bfloat16)
        abf[rows, :] = a_bf
        d = jax.lax.rsqrt(jnp.sum(a, axis=1, keepdims=True) + 1.0)
        dsc[rows, :] = d
        z = jnp.dot(x_ref[...].astype(jnp.bfloat16),
                    w0_ref[...].astype(jnp.bfloat16),
                    preferred_element_type=jnp.float32)
        u0 = (d * z).astype(jnp.bfloat16)
        part = jax.lax.dot_general(
            a_bf, u0, (((0,), (0,)), ((), ())),
            preferred_element_type=jnp.float32)

        @pl.when(m == 0)
        def _():
            acc[...] = part

        @pl.when(m > 0)
        def _():
            acc[...] += part

    @pl.when(s == 1)
    def _layer0():
        d = dsc[rows, :]
        out = d * acc[rows, :] + b0_ref[...]
        h = jnp.maximum(out, 0.0).astype(jnp.bfloat16)
        z = jnp.dot(h, w1_ref[...].astype(jnp.bfloat16),
                    preferred_element_type=jnp.float32)
        ua[rows, :] = (d * z).astype(jnp.bfloat16)

    @pl.when(s == 2)
    def _layer1():
        agg = jnp.dot(abf[rows, :], ua[...],
                      preferred_element_type=jnp.float32)
        d = dsc[rows, :]
        out = d * agg + b1_ref[...]
        h = jnp.maximum(out, 0.0).astype(jnp.bfloat16)
        z = jnp.dot(h, w2_ref[...].astype(jnp.bfloat16),
                    preferred_element_type=jnp.float32)
        ub[rows, :f_out] = (d * z).astype(jnp.bfloat16)

    @pl.when(s == 3)
    def _layer2():
        agg = jnp.dot(abf[rows, :], ub[:, :f_out],
                      preferred_element_type=jnp.float32)
        o_ref[...] = dsc[rows, :] * agg + b2_ref[...]


def kernel(x, adj, w_0, b_0, w_1, b_1, w_2, b_2):
    n, f_in = x.shape
    f_h = w_1.shape[0]
    f_out = w_2.shape[1]
    tm = _ROW_TILE
    nstrips = n // tm
    num_stages = 4

    last = nstrips - 1
    adj_idx = lambda s, m: (jax.lax.select(s == 0, m, last), 0)
    out_idx = lambda s, m: (jax.lax.select(s == num_stages - 1, m, 0), 0)
    const = lambda s, m: (0, 0)

    return pl.pallas_call(
        functools.partial(_gcn_kernel, f_out=f_out),
        grid=(num_stages, nstrips),
        in_specs=[
            pl.BlockSpec((tm, n), adj_idx),
            pl.BlockSpec((tm, f_in), adj_idx),
            pl.BlockSpec((f_in, f_h), const),
            pl.BlockSpec((f_h, f_h), const),
            pl.BlockSpec((f_h, f_out), const),
            pl.BlockSpec((1, f_h), const),
            pl.BlockSpec((1, f_h), const),
            pl.BlockSpec((1, f_out), const),
        ],
        out_specs=pl.BlockSpec((tm, f_out), out_idx),
        out_shape=jax.ShapeDtypeStruct((n, f_out), jnp.float32),
        scratch_shapes=[
            pltpu.VMEM((n, n), jnp.bfloat16),
            pltpu.VMEM((n, f_h), jnp.bfloat16),
            pltpu.VMEM((n, f_h), jnp.bfloat16),
            pltpu.VMEM((n, 1), jnp.float32),
            pltpu.VMEM((n, f_h), jnp.float32),
        ],
        compiler_params=pltpu.CompilerParams(
            dimension_semantics=("arbitrary", "arbitrary"),
            vmem_limit_bytes=_VMEM_LIMIT,
        ),
    )(adj, x, w_0, w_1, w_2,
      b_0.reshape(1, -1), b_1.reshape(1, -1), b_2.reshape(1, -1))
